# Initial kernel scaffold; baseline (speedup 1.0000x reference)
import jax
import jax.numpy as jnp
from jax.experimental import pallas as pl


def kernel(inputs, we):
    raise NotImplementedError("write your pallas kernel here")


# speedup vs baseline: 1.3487x; 1.3487x over previous
import functools

import jax
import jax.numpy as jnp
from jax import lax
from jax.experimental import pallas as pl
from jax.experimental.pallas import tpu as pltpu
from jax.experimental.pallas import tpu_sc as plsc

B, L, K = 1024, 200, 3
D = 64
R = B * L
NC, NS = 2, 16
NW = NC * NS
ROWS_PER_W = R // NW
CHUNK = 256
NCHUNK = ROWS_PER_W // CHUNK


def _sc_embed(we, idx):
    mesh = plsc.VectorSubcoreMesh(core_axis_name="c", subcore_axis_name="s")

    @functools.partial(
        pl.kernel,
        mesh=mesh,
        out_type=jax.ShapeDtypeStruct((R, D), jnp.float32),
        scratch_types=[
            pltpu.VMEM((K * CHUNK,), jnp.int32),
            pltpu.VMEM((K * CHUNK, D), jnp.float32),
            pltpu.VMEM((CHUNK, D), jnp.float32),
            pltpu.SemaphoreType.DMA,
        ],
        compiler_params=pltpu.CompilerParams(use_tc_tiling_on_sc=False),
    )
    def k(we_hbm, idx_hbm, out_hbm, idx_v, rows_v, out_v, sem):
        wid = lax.axis_index("s") * NC + lax.axis_index("c")
        base = wid * ROWS_PER_W

        def chunk_body(cc, _):
            row0 = base + cc * CHUNK
            pltpu.sync_copy(idx_hbm.at[pl.ds(row0 * K, K * CHUNK)], idx_v)
            pltpu.async_copy(we_hbm.at[idx_v], rows_v, sem).wait()

            def row_body(i, _):
                for v in range(D // 16):
                    s = pl.ds(v * 16, 16)
                    out_v[i, s] = (
                        rows_v[K * i, s]
                        + rows_v[K * i + 1, s]
                        + rows_v[K * i + 2, s]
                    )
                return 0

            lax.fori_loop(0, CHUNK, row_body, 0)
            pltpu.sync_copy(out_v, out_hbm.at[pl.ds(row0, CHUNK)])
            return 0

        lax.fori_loop(0, NCHUNK, chunk_body, 0)

    return k(we, idx)


def kernel(inputs, we):
    idx = inputs.reshape(-1).astype(jnp.int32)
    out = _sc_embed(we, idx)
    return out.reshape(B, L, D)


# speedup vs baseline: 1.4835x; 1.0999x over previous
import functools

import jax
import jax.numpy as jnp
from jax import lax
from jax.experimental import pallas as pl
from jax.experimental.pallas import tpu as pltpu
from jax.experimental.pallas import tpu_sc as plsc

B, L, K = 1024, 200, 3
D = 64
R = B * L
NC, NS = 2, 16
NW = NC * NS
ROWS_PER_W = R // NW
CHUNK = 200
CK = CHUNK * K
NCHUNK = ROWS_PER_W // CHUNK


def _sc_embed(we, idx):
    mesh = plsc.VectorSubcoreMesh(core_axis_name="c", subcore_axis_name="s")

    @functools.partial(
        pl.kernel,
        mesh=mesh,
        out_type=jax.ShapeDtypeStruct((R, D), jnp.float32),
        scratch_types=[
            pltpu.VMEM((ROWS_PER_W * K,), jnp.int32),
            pltpu.VMEM((CK, D), jnp.float32),
            pltpu.VMEM((CK, D), jnp.float32),
            pltpu.VMEM((CHUNK, D), jnp.float32),
            pltpu.VMEM((CHUNK, D), jnp.float32),
            pltpu.SemaphoreType.DMA,
            pltpu.SemaphoreType.DMA,
            pltpu.SemaphoreType.DMA,
            pltpu.SemaphoreType.DMA,
        ],
        compiler_params=pltpu.CompilerParams(use_tc_tiling_on_sc=False),
    )
    def k(we_hbm, idx_hbm, out_hbm, idx_all, rows0, rows1, outv0, outv1,
          gsem0, gsem1, wsem0, wsem1):
        rows = (rows0, rows1)
        outv = (outv0, outv1)
        gsem = (gsem0, gsem1)
        wsem = (wsem0, wsem1)

        wid = lax.axis_index("s") * NC + lax.axis_index("c")
        base = wid * ROWS_PER_W

        pltpu.sync_copy(idx_hbm.at[pl.ds(base * K, ROWS_PER_W * K)], idx_all)

        def gather_copy(cc, b):
            return pltpu.make_async_copy(
                we_hbm.at[idx_all.at[pl.ds(cc * CK, CK)]], rows[b], gsem[b])

        def out_copy(cc, b):
            return pltpu.make_async_copy(
                outv[b], out_hbm.at[pl.ds(base + cc * CHUNK, CHUNK)], wsem[b])

        gather_copy(0, 0).start()

        def step(cc, b):
            @pl.when(cc + 1 < NCHUNK)
            def _():
                gather_copy(cc + 1, 1 - b).start()

            gather_copy(cc, b).wait()

            @pl.when(cc >= 2)
            def _():
                out_copy(cc - 2, b).wait()

            rv = rows[b]
            ov = outv[b]

            def row_body(i, _):
                for v in range(D // 16):
                    s = pl.ds(v * 16, 16)
                    ov[i, s] = rv[K * i, s] + rv[K * i + 1, s] + rv[K * i + 2, s]
                return 0

            lax.fori_loop(0, CHUNK, row_body, 0)
            out_copy(cc, b).start()

        def pair_body(g, _):
            step(2 * g, 0)
            step(2 * g + 1, 1)
            return 0

        lax.fori_loop(0, NCHUNK // 2, pair_body, 0)
        out_copy(NCHUNK - 2, 0).wait()
        out_copy(NCHUNK - 1, 1).wait()

    return k(we, idx)


def kernel(inputs, we):
    idx = inputs.reshape(-1).astype(jnp.int32)
    out = _sc_embed(we, idx)
    return out.reshape(B, L, D)


# speedup vs baseline: 1.5532x; 1.0470x over previous
import functools

import jax
import jax.numpy as jnp
from jax import lax
from jax.experimental import pallas as pl
from jax.experimental.pallas import tpu as pltpu
from jax.experimental.pallas import tpu_sc as plsc

B, L, K = 1024, 200, 3
D = 64
NC, NS = 2, 16
NW = NC * NS
B_PER_W = B // NW


def _sc_embed(we, idx):
    mesh = plsc.VectorSubcoreMesh(core_axis_name="c", subcore_axis_name="s")

    @functools.partial(
        pl.kernel,
        mesh=mesh,
        out_type=jax.ShapeDtypeStruct((B, L, D), jnp.float32),
        scratch_types=[
            pltpu.VMEM((B_PER_W, L * K), jnp.int32),
            pltpu.VMEM((L * K, D), jnp.float32),
            pltpu.VMEM((L * K, D), jnp.float32),
            pltpu.VMEM((L, D), jnp.float32),
            pltpu.VMEM((L, D), jnp.float32),
            pltpu.SemaphoreType.DMA,
            pltpu.SemaphoreType.DMA,
            pltpu.SemaphoreType.DMA,
            pltpu.SemaphoreType.DMA,
        ],
        compiler_params=pltpu.CompilerParams(use_tc_tiling_on_sc=False),
    )
    def k(we_hbm, idx_hbm, out_hbm, idx_all, rows0, rows1, outv0, outv1,
          gsem0, gsem1, wsem0, wsem1):
        rows = (rows0, rows1)
        outv = (outv0, outv1)
        gsem = (gsem0, gsem1)
        wsem = (wsem0, wsem1)

        wid = lax.axis_index("s") * NC + lax.axis_index("c")
        base = wid * B_PER_W

        pltpu.sync_copy(idx_hbm.at[pl.ds(base, B_PER_W)], idx_all)

        def gather_copy(cc, b):
            return pltpu.make_async_copy(
                we_hbm.at[idx_all.at[cc]], rows[b], gsem[b])

        def out_copy(cc, b):
            return pltpu.make_async_copy(outv[b], out_hbm.at[base + cc], wsem[b])

        gather_copy(0, 0).start()

        def step(cc, b):
            @pl.when(cc + 1 < B_PER_W)
            def _():
                gather_copy(cc + 1, 1 - b).start()

            gather_copy(cc, b).wait()

            @pl.when(cc >= 2)
            def _():
                out_copy(cc - 2, b).wait()

            rv = rows[b]
            ov = outv[b]

            def row_body(i, _):
                for v in range(D // 16):
                    s = pl.ds(v * 16, 16)
                    ov[i, s] = rv[K * i, s] + rv[K * i + 1, s] + rv[K * i + 2, s]
                return 0

            lax.fori_loop(0, L, row_body, 0)
            out_copy(cc, b).start()

        def pair_body(g, _):
            step(2 * g, 0)
            step(2 * g + 1, 1)
            return 0

        lax.fori_loop(0, B_PER_W // 2, pair_body, 0)
        out_copy(B_PER_W - 2, 0).wait()
        out_copy(B_PER_W - 1, 1).wait()

    return k(we, idx)


def kernel(inputs, we):
    return _sc_embed(we, inputs.astype(jnp.int32).reshape(B, L * K))
